# Initial kernel scaffold; baseline (speedup 1.0000x reference)
import jax
import jax.numpy as jnp
from jax.experimental import pallas as pl


def kernel(positions, pe_weight):
    raise NotImplementedError("write your pallas kernel here")


# speedup vs baseline: 1.3425x; 1.3425x over previous
import functools

import jax
import jax.numpy as jnp
from jax import lax
from jax.experimental import pallas as pl
from jax.experimental.pallas import tpu as pltpu
from jax.experimental.pallas import tpu_sc as plsc

D_MODEL = 128
MAX_LEN = 100000
N = 1024 * 200

_NC, _NS = 2, 16
_NW = _NC * _NS
_ROWS_PER_W = N // _NW
_CHUNK = 256
_NCHUNK = _ROWS_PER_W // _CHUNK

_MESH = plsc.VectorSubcoreMesh(core_axis_name="c", subcore_axis_name="s")


@functools.partial(
    pl.kernel,
    out_type=jax.ShapeDtypeStruct((N, D_MODEL), jnp.float32),
    mesh=_MESH,
    scratch_types=[
        pltpu.VMEM((_ROWS_PER_W,), jnp.int32),
        pltpu.VMEM((_CHUNK, D_MODEL), jnp.float32),
        pltpu.SemaphoreType.DMA,
    ],
)
def _gather_rows(table_hbm, idx_hbm, out_hbm, idx_v, rows_v, sem):
    wid = lax.axis_index("s") * _NC + lax.axis_index("c")
    base = wid * _ROWS_PER_W
    pltpu.sync_copy(idx_hbm.at[pl.ds(base, _ROWS_PER_W)], idx_v)

    def body(c, _):
        pltpu.async_copy(
            table_hbm.at[idx_v.at[pl.ds(c * _CHUNK, _CHUNK)]], rows_v, sem
        ).wait()
        pltpu.sync_copy(rows_v, out_hbm.at[pl.ds(base + c * _CHUNK, _CHUNK)])
        return 0

    lax.fori_loop(0, _NCHUNK, body, 0)


def kernel(positions, pe_weight):
    b, s = positions.shape
    flat = positions.reshape(-1).astype(jnp.int32)
    i = jnp.arange(N, dtype=jnp.int32)
    fp = jnp.full((MAX_LEN,), N, dtype=jnp.int32).at[flat].min(i)
    g = fp[flat]
    cum = jnp.cumsum(g == i).astype(jnp.int32) - 1
    mapped = cum[g]
    out = _gather_rows(pe_weight, mapped)
    return out.reshape(b, s, D_MODEL)


# speedup vs baseline: 1.4145x; 1.0536x over previous
import functools

import jax
import jax.numpy as jnp
from jax import lax
from jax.experimental import pallas as pl
from jax.experimental.pallas import tpu as pltpu
from jax.experimental.pallas import tpu_sc as plsc

D_MODEL = 128
MAX_LEN = 100000
N = 1024 * 200

_NC, _NS = 2, 16
_NW = _NC * _NS
_ROWS_PER_W = N // _NW
_CHUNK = 256
_NCHUNK = _ROWS_PER_W // _CHUNK
_VPW = _ROWS_PER_W // 16

_TPAD = 102400
_STRIPE = _TPAD // _NW
_SENT = jnp.int32(0x0FFFFFFF)

_MESH = plsc.VectorSubcoreMesh(core_axis_name="c", subcore_axis_name="s")
_PARAMS = pltpu.CompilerParams(needs_layout_passes=False)


def _wid():
    return lax.axis_index("s") * _NC + lax.axis_index("c")


@functools.partial(
    pl.kernel,
    out_type=jax.ShapeDtypeStruct((_NW, _TPAD), jnp.int32),
    mesh=_MESH,
    compiler_params=_PARAMS,
    scratch_types=[
        pltpu.VMEM((_TPAD,), jnp.int32),
        pltpu.VMEM((_ROWS_PER_W,), jnp.int32),
    ],
)
def _k1_partial_fp(flat_hbm, part_hbm, table_v, chunk_v):
    wid = _wid()
    base = wid * _ROWS_PER_W
    pltpu.sync_copy(flat_hbm.at[pl.ds(base, _ROWS_PER_W)], chunk_v)

    sentv = jnp.full((16,), _SENT, jnp.int32)

    def init_body(t, _):
        table_v[pl.ds(t * 16, 16)] = sentv
        return 0

    lax.fori_loop(0, _TPAD // 16, init_body, 0)

    lane = lax.iota(jnp.int32, 16)
    shift_idx = jnp.maximum(lane - 1, 0)

    def batch_body(b, _):
        lv = chunk_v[pl.ds(b * 16, 16)]
        li = lane + b * 16
        key = (lv << 13) | li
        ks, _unused = plsc.sort_key_val(key, li)
        sv = lax.shift_right_logical(ks, 13)
        sli = ks & 8191
        prev = sv.at[shift_idx].get(mode="promise_in_bounds")
        first = (lane == 0) | (sv != prev)
        r = plsc.load_gather(table_v, [sv])
        m = first & (sli < r)
        plsc.store_scatter(table_v, [sv], sli, mask=m)
        return 0

    lax.fori_loop(0, _VPW, batch_body, 0)

    def conv_body(t, _):
        e = table_v[pl.ds(t * 16, 16)]
        table_v[pl.ds(t * 16, 16)] = jnp.where(e == _SENT, N, e + base)
        return 0

    lax.fori_loop(0, _TPAD // 16, conv_body, 0)
    pltpu.sync_copy(table_v, part_hbm.at[wid])


@functools.partial(
    pl.kernel,
    out_type=jax.ShapeDtypeStruct((_TPAD,), jnp.int32),
    mesh=_MESH,
    compiler_params=_PARAMS,
    scratch_types=[
        pltpu.VMEM((_STRIPE,), jnp.int32),
        pltpu.VMEM((_STRIPE,), jnp.int32),
    ],
)
def _k2_combine_fp(part_hbm, fp_hbm, acc_v, cur_v):
    wid = _wid()
    base = wid * _STRIPE
    pltpu.sync_copy(part_hbm.at[0, pl.ds(base, _STRIPE)], acc_v)

    def row_body(r, _):
        pltpu.sync_copy(part_hbm.at[r, pl.ds(base, _STRIPE)], cur_v)

        def vec_body(t, _):
            sl = pl.ds(t * 16, 16)
            acc_v[sl] = jnp.minimum(acc_v[sl], cur_v[sl])
            return 0

        lax.fori_loop(0, _STRIPE // 16, vec_body, 0)
        return 0

    lax.fori_loop(1, _NW, row_body, 0)
    pltpu.sync_copy(acc_v, fp_hbm.at[pl.ds(base, _STRIPE)])


@functools.partial(
    pl.kernel,
    out_type=jax.ShapeDtypeStruct((N, D_MODEL), jnp.float32),
    mesh=_MESH,
    scratch_types=[
        pltpu.VMEM((_ROWS_PER_W,), jnp.int32),
        pltpu.VMEM((_CHUNK, D_MODEL), jnp.float32),
        pltpu.SemaphoreType.DMA,
    ],
)
def _gather_rows(table_hbm, idx_hbm, out_hbm, idx_v, rows_v, sem):
    base = _wid() * _ROWS_PER_W
    pltpu.sync_copy(idx_hbm.at[pl.ds(base, _ROWS_PER_W)], idx_v)

    def body(c, _):
        pltpu.async_copy(
            table_hbm.at[idx_v.at[pl.ds(c * _CHUNK, _CHUNK)]], rows_v, sem
        ).wait()
        pltpu.sync_copy(rows_v, out_hbm.at[pl.ds(base + c * _CHUNK, _CHUNK)])
        return 0

    lax.fori_loop(0, _NCHUNK, body, 0)


def kernel(positions, pe_weight):
    b, s = positions.shape
    flat = positions.reshape(-1).astype(jnp.int32)
    part = _k1_partial_fp(flat)
    fp = _k2_combine_fp(part)[:MAX_LEN]
    i = jnp.arange(N, dtype=jnp.int32)
    g = fp[flat]
    cum = jnp.cumsum(g == i).astype(jnp.int32) - 1
    mapped = cum[g]
    out = _gather_rows(pe_weight, mapped)
    return out.reshape(b, s, D_MODEL)


# speedup vs baseline: 8.4494x; 5.9734x over previous
import functools

import jax
import jax.numpy as jnp
from jax import lax
from jax.experimental import pallas as pl
from jax.experimental.pallas import tpu as pltpu
from jax.experimental.pallas import tpu_sc as plsc

D_MODEL = 128
MAX_LEN = 100000
N = 1024 * 200

_NC, _NS = 2, 16
_NW = _NC * _NS
_ROWS_PER_W = N // _NW
_CHUNK = 256
_NCHUNK = _ROWS_PER_W // _CHUNK
_VPW = _ROWS_PER_W // 16

_TPAD = 102400
_STRIPE = _TPAD // _NW
_SENT = jnp.int32(0x0FFFFFFF)

_MESH = plsc.VectorSubcoreMesh(core_axis_name="c", subcore_axis_name="s")
_PARAMS = pltpu.CompilerParams(needs_layout_passes=False)


def _wid():
    return lax.axis_index("s") * _NC + lax.axis_index("c")


@functools.partial(
    pl.kernel,
    out_type=jax.ShapeDtypeStruct((_NW, _TPAD), jnp.int32),
    mesh=_MESH,
    compiler_params=_PARAMS,
    scratch_types=[
        pltpu.VMEM((_TPAD,), jnp.int32),
        pltpu.VMEM((_ROWS_PER_W,), jnp.int32),
    ],
)
def _k1_partial_fp(flat_hbm, part_hbm, table_v, chunk_v):
    wid = _wid()
    base = wid * _ROWS_PER_W
    pltpu.sync_copy(flat_hbm.at[pl.ds(base, _ROWS_PER_W)], chunk_v)

    sentv = jnp.full((16,), _SENT, jnp.int32)

    def init_body(t, _):
        table_v[pl.ds(t * 16, 16)] = sentv
        return 0

    lax.fori_loop(0, _TPAD // 16, init_body, 0)

    lane = lax.iota(jnp.int32, 16)
    shift_idx = jnp.maximum(lane - 1, 0)

    def batch_body(b, _):
        lv = chunk_v[pl.ds(b * 16, 16)]
        li = lane + b * 16
        key = (lv << 13) | li
        ks, _unused = plsc.sort_key_val(key, li)
        sv = lax.shift_right_logical(ks, 13)
        sli = ks & 8191
        prev = sv.at[shift_idx].get(mode="promise_in_bounds")
        first = (lane == 0) | (sv != prev)
        r = plsc.load_gather(table_v, [sv])
        m = first & (sli < r)
        plsc.store_scatter(table_v, [sv], sli, mask=m)
        return 0

    lax.fori_loop(0, _VPW, batch_body, 0)

    def conv_body(t, _):
        e = table_v[pl.ds(t * 16, 16)]
        table_v[pl.ds(t * 16, 16)] = jnp.where(e == _SENT, N, e + base)
        return 0

    lax.fori_loop(0, _TPAD // 16, conv_body, 0)
    pltpu.sync_copy(table_v, part_hbm.at[wid])


@functools.partial(
    pl.kernel,
    out_type=jax.ShapeDtypeStruct((_TPAD,), jnp.int32),
    mesh=_MESH,
    compiler_params=_PARAMS,
    scratch_types=[
        pltpu.VMEM((_STRIPE,), jnp.int32),
        pltpu.VMEM((_STRIPE,), jnp.int32),
    ],
)
def _k2_combine_fp(part_hbm, fp_hbm, acc_v, cur_v):
    wid = _wid()
    base = wid * _STRIPE
    pltpu.sync_copy(part_hbm.at[0, pl.ds(base, _STRIPE)], acc_v)

    def row_body(r, _):
        pltpu.sync_copy(part_hbm.at[r, pl.ds(base, _STRIPE)], cur_v)

        def vec_body(t, _):
            sl = pl.ds(t * 16, 16)
            acc_v[sl] = jnp.minimum(acc_v[sl], cur_v[sl])
            return 0

        lax.fori_loop(0, _STRIPE // 16, vec_body, 0)
        return 0

    lax.fori_loop(1, _NW, row_body, 0)
    pltpu.sync_copy(acc_v, fp_hbm.at[pl.ds(base, _STRIPE)])


@functools.partial(
    pl.kernel,
    out_type=(
        jax.ShapeDtypeStruct((N,), jnp.int32),
        jax.ShapeDtypeStruct((N,), jnp.int32),
    ),
    mesh=_MESH,
    compiler_params=_PARAMS,
    scratch_types=[
        pltpu.VMEM((_ROWS_PER_W,), jnp.int32),
        pltpu.VMEM((_ROWS_PER_W,), jnp.int32),
        pltpu.VMEM((_ROWS_PER_W,), jnp.int32),
        pltpu.SemaphoreType.DMA,
    ],
)
def _k3_local_cumsum(flat_hbm, fp_hbm, g_hbm, lc_hbm, f_v, g_v, lc_v, sem):
    base = _wid() * _ROWS_PER_W
    pltpu.sync_copy(flat_hbm.at[pl.ds(base, _ROWS_PER_W)], f_v)

    def dma_body(c, _):
        sl = pl.ds(c * _CHUNK, _CHUNK)
        pltpu.async_copy(fp_hbm.at[f_v.at[sl]], g_v.at[sl], sem).wait()
        return 0

    lax.fori_loop(0, _NCHUNK, dma_body, 0)

    lane = lax.iota(jnp.int32, 16)

    def vec_body(b, carry):
        sl = pl.ds(b * 16, 16)
        gv = g_v[sl]
        isf = (gv == lane + (base + b * 16)).astype(jnp.int32)
        lc_v[sl] = plsc.cumsum(isf) + carry
        return carry + jnp.sum(isf)

    lax.fori_loop(0, _VPW, vec_body, jnp.int32(0))
    pltpu.sync_copy(g_v, g_hbm.at[pl.ds(base, _ROWS_PER_W)])
    pltpu.sync_copy(lc_v, lc_hbm.at[pl.ds(base, _ROWS_PER_W)])


@functools.partial(
    pl.kernel,
    out_type=jax.ShapeDtypeStruct((N, D_MODEL), jnp.float32),
    mesh=_MESH,
    compiler_params=_PARAMS,
    scratch_types=[
        pltpu.VMEM((_ROWS_PER_W,), jnp.int32),
        pltpu.VMEM((_ROWS_PER_W,), jnp.int32),
        pltpu.VMEM((_ROWS_PER_W,), jnp.int32),
        pltpu.VMEM((32,), jnp.int32),
        pltpu.VMEM((32,), jnp.int32),
        pltpu.VMEM((32,), jnp.int32),
        pltpu.VMEM((_CHUNK, D_MODEL), jnp.float32),
        pltpu.SemaphoreType.DMA,
    ],
)
def _k4_map_and_gather(
    g_hbm, lc_hbm, pe_hbm, out_hbm,
    g_v, lcg_v, m_v, eidx_v, cnt_v, off_v, rows_v, sem,
):
    base = _wid() * _ROWS_PER_W
    pltpu.sync_copy(g_hbm.at[pl.ds(base, _ROWS_PER_W)], g_v)

    lane = lax.iota(jnp.int32, 16)
    eidx_v[pl.ds(0, 16)] = lane * _ROWS_PER_W + (_ROWS_PER_W - 1)
    eidx_v[pl.ds(16, 16)] = (lane + 16) * _ROWS_PER_W + (_ROWS_PER_W - 1)
    pltpu.async_copy(lc_hbm.at[eidx_v], cnt_v, sem).wait()
    c0 = cnt_v[pl.ds(0, 16)]
    c1 = cnt_v[pl.ds(16, 16)]
    off_v[pl.ds(0, 16)] = plsc.cumsum(c0) - c0
    off_v[pl.ds(16, 16)] = plsc.cumsum(c1) - c1 + jnp.sum(c0)

    def dma_body(c, _):
        sl = pl.ds(c * _CHUNK, _CHUNK)
        pltpu.async_copy(lc_hbm.at[g_v.at[sl]], lcg_v.at[sl], sem).wait()
        return 0

    lax.fori_loop(0, _NCHUNK, dma_body, 0)

    def vec_body(b, _):
        sl = pl.ds(b * 16, 16)
        gv = g_v[sl]
        q = lax.shift_right_logical((lax.shift_right_logical(gv, 8)) * 1311, 15)
        offe = plsc.load_gather(off_v, [q])
        m_v[sl] = lcg_v[sl] + offe - 1
        return 0

    lax.fori_loop(0, _VPW, vec_body, 0)

    def row_body(c, _):
        pltpu.async_copy(
            pe_hbm.at[m_v.at[pl.ds(c * _CHUNK, _CHUNK)]], rows_v, sem
        ).wait()
        pltpu.sync_copy(rows_v, out_hbm.at[pl.ds(base + c * _CHUNK, _CHUNK)])
        return 0

    lax.fori_loop(0, _NCHUNK, row_body, 0)


@functools.partial(
    pl.kernel,
    out_type=jax.ShapeDtypeStruct((N, D_MODEL), jnp.float32),
    mesh=_MESH,
    scratch_types=[
        pltpu.VMEM((_ROWS_PER_W,), jnp.int32),
        pltpu.VMEM((_CHUNK, D_MODEL), jnp.float32),
        pltpu.SemaphoreType.DMA,
    ],
)
def _gather_rows(table_hbm, idx_hbm, out_hbm, idx_v, rows_v, sem):
    base = _wid() * _ROWS_PER_W
    pltpu.sync_copy(idx_hbm.at[pl.ds(base, _ROWS_PER_W)], idx_v)

    def body(c, _):
        pltpu.async_copy(
            table_hbm.at[idx_v.at[pl.ds(c * _CHUNK, _CHUNK)]], rows_v, sem
        ).wait()
        pltpu.sync_copy(rows_v, out_hbm.at[pl.ds(base + c * _CHUNK, _CHUNK)])
        return 0

    lax.fori_loop(0, _NCHUNK, body, 0)


def kernel(positions, pe_weight):
    b, s = positions.shape
    flat = positions.reshape(-1).astype(jnp.int32)
    part = _k1_partial_fp(flat)
    fp = _k2_combine_fp(part)
    g, lc = _k3_local_cumsum(flat, fp)
    out = _k4_map_and_gather(g, lc, pe_weight)
    return out.reshape(b, s, D_MODEL)


# speedup vs baseline: 13.4533x; 1.5922x over previous
import functools

import jax
import jax.numpy as jnp
from jax import lax
from jax.experimental import pallas as pl
from jax.experimental.pallas import tpu as pltpu
from jax.experimental.pallas import tpu_sc as plsc

D_MODEL = 128
MAX_LEN = 100000
N = 1024 * 200

_NC, _NS = 2, 16
_NW = _NC * _NS
_ROWS_PER_W = N // _NW
_CHUNK = 256
_NCHUNK = _ROWS_PER_W // _CHUNK
_VPW = _ROWS_PER_W // 16

_RCHUNK = 320
_NRCHUNK = _ROWS_PER_W // _RCHUNK

_TPAD = 102400
_STRIPE = _TPAD // _NW
_SENT = jnp.int32(0x0FFFFFFF)

_MESH = plsc.VectorSubcoreMesh(core_axis_name="c", subcore_axis_name="s")
_PARAMS = pltpu.CompilerParams(needs_layout_passes=False)


def _wid():
    return lax.axis_index("s") * _NC + lax.axis_index("c")


@functools.partial(
    pl.kernel,
    out_type=jax.ShapeDtypeStruct((_NW, _TPAD), jnp.int32),
    mesh=_MESH,
    compiler_params=_PARAMS,
    scratch_types=[
        pltpu.VMEM((_TPAD,), jnp.int32),
        pltpu.VMEM((_ROWS_PER_W,), jnp.int32),
    ],
)
def _k1_partial_fp(flat_hbm, part_hbm, table_v, chunk_v):
    wid = _wid()
    base = wid * _ROWS_PER_W
    pltpu.sync_copy(flat_hbm.at[pl.ds(base, _ROWS_PER_W)], chunk_v)

    sentv = jnp.full((16,), _SENT, jnp.int32)

    def init_body(t, _):
        for u in range(8):
            table_v[pl.ds((t * 8 + u) * 16, 16)] = sentv
        return 0

    lax.fori_loop(0, _TPAD // 128, init_body, 0)

    lane = lax.iota(jnp.int32, 16)
    shift_idx = jnp.maximum(lane - 1, 0)

    def one_batch(b):
        lv = chunk_v[pl.ds(b * 16, 16)]
        li = lane + b * 16
        key = (lv << 13) | li
        ks, sli = plsc.sort_key_val(key, li)
        sv = lax.shift_right_logical(ks, 13)
        prev = sv.at[shift_idx].get(mode="promise_in_bounds")
        first = (lane == 0) | (sv != prev)
        r = plsc.load_gather(table_v, [sv])
        m = first & (sli < r)
        plsc.store_scatter(table_v, [sv], sli, mask=m)

    def batch_body(b, _):
        one_batch(2 * b)
        one_batch(2 * b + 1)
        return 0

    lax.fori_loop(0, _VPW // 2, batch_body, 0)
    pltpu.sync_copy(table_v, part_hbm.at[wid])


@functools.partial(
    pl.kernel,
    out_type=jax.ShapeDtypeStruct((_TPAD,), jnp.int32),
    mesh=_MESH,
    compiler_params=_PARAMS,
    scratch_types=[
        pltpu.VMEM((_STRIPE,), jnp.int32),
        pltpu.VMEM((_STRIPE,), jnp.int32),
        pltpu.VMEM((_STRIPE,), jnp.int32),
        pltpu.SemaphoreType.DMA,
        pltpu.SemaphoreType.DMA,
    ],
)
def _k2_combine_fp(part_hbm, fp_hbm, acc_v, b0_v, b1_v, sem0, sem1):
    base = _wid() * _STRIPE
    sl = pl.ds(base, _STRIPE)
    pltpu.sync_copy(part_hbm.at[0, sl], acc_v)
    pltpu.async_copy(part_hbm.at[1, sl], b0_v, sem0).wait()

    def combine(cur_v, r):
        delta = r * _ROWS_PER_W

        def vec_body(t, _):
            for u in range(8):
                s = pl.ds((t * 8 + u) * 16, 16)
                acc_v[s] = jnp.minimum(acc_v[s], cur_v[s] + delta)
            return 0

        lax.fori_loop(0, _STRIPE // 128, vec_body, 0)

    def row_body(p, _):
        r = 2 * p + 1
        pltpu.async_copy(part_hbm.at[r + 1, sl], b1_v, sem1).wait()
        combine(b0_v, r)

        @pl.when(r + 2 < _NW)
        def _():
            pltpu.async_copy(part_hbm.at[r + 2, sl], b0_v, sem0).wait()

        combine(b1_v, r + 1)
        return 0

    lax.fori_loop(0, (_NW - 2) // 2, row_body, 0)
    combine(b0_v, _NW - 1)
    pltpu.sync_copy(acc_v, fp_hbm.at[sl])


@functools.partial(
    pl.kernel,
    out_type=(
        jax.ShapeDtypeStruct((N,), jnp.int32),
        jax.ShapeDtypeStruct((N,), jnp.int32),
    ),
    mesh=_MESH,
    compiler_params=_PARAMS,
    scratch_types=[
        pltpu.VMEM((_ROWS_PER_W,), jnp.int32),
        pltpu.VMEM((_ROWS_PER_W,), jnp.int32),
        pltpu.VMEM((_ROWS_PER_W,), jnp.int32),
        pltpu.SemaphoreType.DMA,
        pltpu.SemaphoreType.DMA,
    ],
)
def _k3_local_cumsum(flat_hbm, fp_hbm, g_hbm, lc_hbm, f_v, g_v, lc_v, sem0, sem1):
    base = _wid() * _ROWS_PER_W
    pltpu.sync_copy(flat_hbm.at[pl.ds(base, _ROWS_PER_W)], f_v)

    def gsl(c):
        return pl.ds(c * _CHUNK, _CHUNK)

    pltpu.async_copy(fp_hbm.at[f_v.at[gsl(0)]], g_v.at[gsl(0)], sem0)

    def dma_body(p, _):
        c = 2 * p
        pltpu.async_copy(fp_hbm.at[f_v.at[gsl(c + 1)]], g_v.at[gsl(c + 1)], sem1)
        pltpu.make_async_copy(fp_hbm.at[f_v.at[gsl(c)]], g_v.at[gsl(c)], sem0).wait()

        @pl.when(c + 2 < _NCHUNK)
        def _():
            pltpu.async_copy(fp_hbm.at[f_v.at[gsl(c + 2)]], g_v.at[gsl(c + 2)], sem0)

        pltpu.make_async_copy(
            fp_hbm.at[f_v.at[gsl(c + 1)]], g_v.at[gsl(c + 1)], sem1
        ).wait()
        return 0

    lax.fori_loop(0, _NCHUNK // 2, dma_body, 0)
    c_last = _NCHUNK - 1
    pltpu.make_async_copy(
        fp_hbm.at[f_v.at[gsl(c_last)]], g_v.at[gsl(c_last)], sem0
    ).wait()

    lane = lax.iota(jnp.int32, 16)

    def vec_body(b, carry):
        sl = pl.ds(b * 16, 16)
        isf = (g_v[sl] == lane + (base + b * 16)).astype(jnp.int32)
        lc = plsc.cumsum(isf) + carry
        lc_v[sl] = lc
        return carry + jnp.sum(isf)

    lax.fori_loop(0, _VPW, vec_body, jnp.int32(0))
    pltpu.sync_copy(g_v, g_hbm.at[pl.ds(base, _ROWS_PER_W)])
    pltpu.sync_copy(lc_v, lc_hbm.at[pl.ds(base, _ROWS_PER_W)])


@functools.partial(
    pl.kernel,
    out_type=jax.ShapeDtypeStruct((N, D_MODEL), jnp.float32),
    mesh=_MESH,
    compiler_params=_PARAMS,
    scratch_types=[
        pltpu.VMEM((_ROWS_PER_W,), jnp.int32),
        pltpu.VMEM((_ROWS_PER_W,), jnp.int32),
        pltpu.VMEM((32,), jnp.int32),
        pltpu.VMEM((32,), jnp.int32),
        pltpu.VMEM((32,), jnp.int32),
        pltpu.VMEM((_RCHUNK, D_MODEL), jnp.float32),
        pltpu.VMEM((_RCHUNK, D_MODEL), jnp.float32),
        pltpu.SemaphoreType.DMA,
        pltpu.SemaphoreType.DMA,
    ],
)
def _k4_map_and_gather(
    g_hbm, lc_hbm, pe_hbm, out_hbm,
    g_v, m_v, eidx_v, cnt_v, off_v, r0_v, r1_v, sem0, sem1,
):
    base = _wid() * _ROWS_PER_W
    pltpu.sync_copy(g_hbm.at[pl.ds(base, _ROWS_PER_W)], g_v)

    lane = lax.iota(jnp.int32, 16)
    eidx_v[pl.ds(0, 16)] = lane * _ROWS_PER_W + (_ROWS_PER_W - 1)
    eidx_v[pl.ds(16, 16)] = (lane + 16) * _ROWS_PER_W + (_ROWS_PER_W - 1)
    pltpu.async_copy(lc_hbm.at[eidx_v], cnt_v, sem0).wait()
    c0 = cnt_v[pl.ds(0, 16)]
    c1 = cnt_v[pl.ds(16, 16)]
    off_v[pl.ds(0, 16)] = plsc.cumsum(c0) - c0
    off_v[pl.ds(16, 16)] = plsc.cumsum(c1) - c1 + jnp.sum(c0)

    def gsl(c):
        return pl.ds(c * _CHUNK, _CHUNK)

    pltpu.async_copy(lc_hbm.at[g_v.at[gsl(0)]], m_v.at[gsl(0)], sem0)

    def dma_body(p, _):
        c = 2 * p
        pltpu.async_copy(lc_hbm.at[g_v.at[gsl(c + 1)]], m_v.at[gsl(c + 1)], sem1)
        pltpu.make_async_copy(lc_hbm.at[g_v.at[gsl(c)]], m_v.at[gsl(c)], sem0).wait()

        @pl.when(c + 2 < _NCHUNK)
        def _():
            pltpu.async_copy(lc_hbm.at[g_v.at[gsl(c + 2)]], m_v.at[gsl(c + 2)], sem0)

        pltpu.make_async_copy(
            lc_hbm.at[g_v.at[gsl(c + 1)]], m_v.at[gsl(c + 1)], sem1
        ).wait()
        return 0

    lax.fori_loop(0, _NCHUNK // 2, dma_body, 0)
    c_last = _NCHUNK - 1
    pltpu.make_async_copy(
        lc_hbm.at[g_v.at[gsl(c_last)]], m_v.at[gsl(c_last)], sem0
    ).wait()

    def vec_body(b, _):
        for u in range(2):
            sl = pl.ds((b * 2 + u) * 16, 16)
            gv = g_v[sl]
            q = lax.shift_right_logical(lax.shift_right_logical(gv, 8) * 1311, 15)
            offe = plsc.load_gather(off_v, [q])
            m_v[sl] = m_v[sl] + offe - 1
        return 0

    lax.fori_loop(0, _VPW // 2, vec_body, 0)

    def rsl(c):
        return pl.ds(c * _RCHUNK, _RCHUNK)

    pltpu.async_copy(pe_hbm.at[m_v.at[rsl(0)]], r0_v, sem0)

    def row_body(p, _):
        c = 2 * p
        pltpu.async_copy(pe_hbm.at[m_v.at[rsl(c + 1)]], r1_v, sem1)
        pltpu.make_async_copy(pe_hbm.at[m_v.at[rsl(c)]], r0_v, sem0).wait()
        pltpu.sync_copy(r0_v, out_hbm.at[pl.ds(base + c * _RCHUNK, _RCHUNK)])

        @pl.when(c + 2 < _NRCHUNK)
        def _():
            pltpu.async_copy(pe_hbm.at[m_v.at[rsl(c + 2)]], r0_v, sem0)

        pltpu.make_async_copy(pe_hbm.at[m_v.at[rsl(c + 1)]], r1_v, sem1).wait()
        pltpu.sync_copy(r1_v, out_hbm.at[pl.ds(base + (c + 1) * _RCHUNK, _RCHUNK)])
        return 0

    lax.fori_loop(0, _NRCHUNK // 2, row_body, 0)


def kernel(positions, pe_weight):
    b, s = positions.shape
    flat = positions.reshape(-1).astype(jnp.int32)
    part = _k1_partial_fp(flat)
    fp = _k2_combine_fp(part)
    g, lc = _k3_local_cumsum(flat, fp)
    out = _k4_map_and_gather(g, lc, pe_weight)
    return out.reshape(b, s, D_MODEL)


# speedup vs baseline: 14.7359x; 1.0953x over previous
import functools

import jax
import jax.numpy as jnp
from jax import lax
from jax.experimental import pallas as pl
from jax.experimental.pallas import tpu as pltpu
from jax.experimental.pallas import tpu_sc as plsc

D_MODEL = 128
MAX_LEN = 100000
N = 1024 * 200

_NC, _NS = 2, 16
_NW = _NC * _NS
_ROWS_PER_W = N // _NW
_CHUNK = 256
_NCHUNK = _ROWS_PER_W // _CHUNK
_VPW = _ROWS_PER_W // 16

_RCHUNK = 320
_NRCHUNK = _ROWS_PER_W // _RCHUNK

_TPAD = 102400
_STRIPE = _TPAD // _NW
_SENT = jnp.int32(0x0FFFFFFF)

_MESH = plsc.VectorSubcoreMesh(core_axis_name="c", subcore_axis_name="s")
_PARAMS = pltpu.CompilerParams(needs_layout_passes=False)


def _wid():
    return lax.axis_index("s") * _NC + lax.axis_index("c")


@functools.partial(
    pl.kernel,
    out_type=jax.ShapeDtypeStruct((_NW, _TPAD), jnp.int32),
    mesh=_MESH,
    compiler_params=_PARAMS,
    scratch_types=[
        pltpu.VMEM((_TPAD,), jnp.int32),
        pltpu.VMEM((_ROWS_PER_W,), jnp.int32),
    ],
)
def _k1_partial_fp(flat_hbm, part_hbm, table_v, chunk_v):
    wid = _wid()
    base = wid * _ROWS_PER_W
    pltpu.sync_copy(flat_hbm.at[pl.ds(base, _ROWS_PER_W)], chunk_v)

    sentv = jnp.full((16,), _SENT, jnp.int32)

    def init_body(t, _):
        for u in range(8):
            table_v[pl.ds((t * 8 + u) * 16, 16)] = sentv
        return 0

    lax.fori_loop(0, _TPAD // 128, init_body, 0)

    lane = lax.iota(jnp.int32, 16)
    shift_idx = jnp.maximum(lane - 1, 0)

    def one_batch(b):
        lv = chunk_v[pl.ds(b * 16, 16)]
        li = lane + b * 16
        key = (lv << 13) | li
        ks, sli = plsc.sort_key_val(key, li)
        sv = lax.shift_right_logical(ks, 13)
        prev = sv.at[shift_idx].get(mode="promise_in_bounds")
        first = (lane == 0) | (sv != prev)
        r = plsc.load_gather(table_v, [sv])
        m = first & (sli < r)
        plsc.store_scatter(table_v, [sv], sli, mask=m)

    def batch_body(b, _):
        one_batch(2 * b)
        one_batch(2 * b + 1)
        return 0

    lax.fori_loop(0, _VPW // 2, batch_body, 0)
    pltpu.sync_copy(table_v, part_hbm.at[wid])


@functools.partial(
    pl.kernel,
    out_type=jax.ShapeDtypeStruct((_TPAD,), jnp.int32),
    mesh=_MESH,
    compiler_params=_PARAMS,
    scratch_types=[
        pltpu.VMEM((_STRIPE,), jnp.int32),
        pltpu.VMEM((_STRIPE,), jnp.int32),
        pltpu.VMEM((_STRIPE,), jnp.int32),
        pltpu.SemaphoreType.DMA,
        pltpu.SemaphoreType.DMA,
    ],
)
def _k2_combine_fp(part_hbm, fp_hbm, acc_v, b0_v, b1_v, sem0, sem1):
    base = _wid() * _STRIPE
    sl = pl.ds(base, _STRIPE)
    pltpu.sync_copy(part_hbm.at[0, sl], acc_v)
    pltpu.async_copy(part_hbm.at[1, sl], b0_v, sem0)

    def combine(cur_v, r):
        delta = r * _ROWS_PER_W

        def vec_body(t, _):
            for u in range(8):
                s = pl.ds((t * 8 + u) * 16, 16)
                acc_v[s] = jnp.minimum(acc_v[s], cur_v[s] + delta)
            return 0

        lax.fori_loop(0, _STRIPE // 128, vec_body, 0)

    def row_body(p, _):
        r = 2 * p + 1
        pltpu.async_copy(part_hbm.at[r + 1, sl], b1_v, sem1)
        pltpu.make_async_copy(part_hbm.at[r, sl], b0_v, sem0).wait()
        combine(b0_v, r)
        pltpu.async_copy(part_hbm.at[r + 2, sl], b0_v, sem0)
        pltpu.make_async_copy(part_hbm.at[r + 1, sl], b1_v, sem1).wait()
        combine(b1_v, r + 1)
        return 0

    lax.fori_loop(0, (_NW - 2) // 2, row_body, 0)
    pltpu.make_async_copy(part_hbm.at[_NW - 1, sl], b0_v, sem0).wait()
    combine(b0_v, _NW - 1)
    pltpu.sync_copy(acc_v, fp_hbm.at[sl])


@functools.partial(
    pl.kernel,
    out_type=(
        jax.ShapeDtypeStruct((N,), jnp.int32),
        jax.ShapeDtypeStruct((N,), jnp.int32),
    ),
    mesh=_MESH,
    compiler_params=_PARAMS,
    scratch_types=[
        pltpu.VMEM((_ROWS_PER_W,), jnp.int32),
        pltpu.VMEM((_ROWS_PER_W,), jnp.int32),
        pltpu.VMEM((_ROWS_PER_W,), jnp.int32),
        pltpu.SemaphoreType.DMA,
        pltpu.SemaphoreType.DMA,
    ],
)
def _k3_local_cumsum(flat_hbm, fp_hbm, g_hbm, lc_hbm, f_v, g_v, lc_v, sem0, sem1):
    base = _wid() * _ROWS_PER_W
    pltpu.sync_copy(flat_hbm.at[pl.ds(base, _ROWS_PER_W)], f_v)

    def gsl(c):
        return pl.ds(c * _CHUNK, _CHUNK)

    pltpu.async_copy(fp_hbm.at[f_v.at[gsl(0)]], g_v.at[gsl(0)], sem0)

    def dma_body(p, _):
        c = 2 * p
        pltpu.async_copy(fp_hbm.at[f_v.at[gsl(c + 1)]], g_v.at[gsl(c + 1)], sem1)
        pltpu.make_async_copy(fp_hbm.at[f_v.at[gsl(c)]], g_v.at[gsl(c)], sem0).wait()

        @pl.when(c + 2 < _NCHUNK)
        def _():
            pltpu.async_copy(fp_hbm.at[f_v.at[gsl(c + 2)]], g_v.at[gsl(c + 2)], sem0)

        pltpu.make_async_copy(
            fp_hbm.at[f_v.at[gsl(c + 1)]], g_v.at[gsl(c + 1)], sem1
        ).wait()
        return 0

    lax.fori_loop(0, _NCHUNK // 2, dma_body, 0)
    c_last = _NCHUNK - 1
    pltpu.make_async_copy(
        fp_hbm.at[f_v.at[gsl(c_last)]], g_v.at[gsl(c_last)], sem0
    ).wait()

    lane = lax.iota(jnp.int32, 16)

    def vec_body(b, carry):
        sl = pl.ds(b * 16, 16)
        isf = (g_v[sl] == lane + (base + b * 16)).astype(jnp.int32)
        lc = plsc.cumsum(isf) + carry
        lc_v[sl] = lc
        return carry + jnp.sum(isf)

    lax.fori_loop(0, _VPW, vec_body, jnp.int32(0))
    pltpu.sync_copy(g_v, g_hbm.at[pl.ds(base, _ROWS_PER_W)])
    pltpu.sync_copy(lc_v, lc_hbm.at[pl.ds(base, _ROWS_PER_W)])


@functools.partial(
    pl.kernel,
    out_type=jax.ShapeDtypeStruct((N, D_MODEL), jnp.float32),
    mesh=_MESH,
    compiler_params=_PARAMS,
    scratch_types=[
        pltpu.VMEM((_ROWS_PER_W,), jnp.int32),
        pltpu.VMEM((_ROWS_PER_W,), jnp.int32),
        pltpu.VMEM((32,), jnp.int32),
        pltpu.VMEM((32,), jnp.int32),
        pltpu.VMEM((32,), jnp.int32),
        pltpu.VMEM((_RCHUNK, D_MODEL), jnp.float32),
        pltpu.VMEM((_RCHUNK, D_MODEL), jnp.float32),
        pltpu.SemaphoreType.DMA,
        pltpu.SemaphoreType.DMA,
        pltpu.SemaphoreType.DMA,
    ],
)
def _k4_map_and_gather(
    g_hbm, lc_hbm, pe_hbm, out_hbm,
    g_v, m_v, eidx_v, cnt_v, off_v, r0_v, r1_v, sem0, sem1, semw,
):
    base = _wid() * _ROWS_PER_W
    pltpu.sync_copy(g_hbm.at[pl.ds(base, _ROWS_PER_W)], g_v)

    lane = lax.iota(jnp.int32, 16)
    eidx_v[pl.ds(0, 16)] = lane * _ROWS_PER_W + (_ROWS_PER_W - 1)
    eidx_v[pl.ds(16, 16)] = (lane + 16) * _ROWS_PER_W + (_ROWS_PER_W - 1)
    pltpu.async_copy(lc_hbm.at[eidx_v], cnt_v, sem0).wait()
    c0 = cnt_v[pl.ds(0, 16)]
    c1 = cnt_v[pl.ds(16, 16)]
    off_v[pl.ds(0, 16)] = plsc.cumsum(c0) - c0
    off_v[pl.ds(16, 16)] = plsc.cumsum(c1) - c1 + jnp.sum(c0)

    def rsl(c):
        return pl.ds(c * _RCHUNK, _RCHUNK)

    def wg_issue(c):
        pltpu.async_copy(lc_hbm.at[g_v.at[rsl(c)]], m_v.at[rsl(c)], semw)

    def wg_wait(c):
        pltpu.make_async_copy(
            lc_hbm.at[g_v.at[rsl(c)]], m_v.at[rsl(c)], semw
        ).wait()

    def compute_m(c):
        def vb(b, _):
            sl = pl.ds(c * _RCHUNK + b * 16, 16)
            gv = g_v[sl]
            q = lax.shift_right_logical(lax.shift_right_logical(gv, 8) * 1311, 15)
            offe = plsc.load_gather(off_v, [q])
            m_v[sl] = m_v[sl] + offe - 1
            return 0

        lax.fori_loop(0, _RCHUNK // 16, vb, 0)

    def rg_issue(c, buf, sem):
        pltpu.async_copy(pe_hbm.at[m_v.at[rsl(c)]], buf, sem)

    def rg_wait(c, buf, sem):
        pltpu.make_async_copy(pe_hbm.at[m_v.at[rsl(c)]], buf, sem).wait()

    def store(c, buf):
        pltpu.sync_copy(buf, out_hbm.at[pl.ds(base + c * _RCHUNK, _RCHUNK)])

    wg_issue(0)
    wg_wait(0)
    compute_m(0)
    wg_issue(1)
    rg_issue(0, r0_v, sem0)
    wg_wait(1)
    compute_m(1)
    wg_issue(2)

    def row_body(p, _):
        c = 2 * p
        rg_issue(c + 1, r1_v, sem1)

        @pl.when(c + 2 < _NRCHUNK)
        def _():
            wg_wait(c + 2)
            compute_m(c + 2)

        @pl.when(c + 3 < _NRCHUNK)
        def _():
            wg_issue(c + 3)

        rg_wait(c, r0_v, sem0)
        store(c, r0_v)

        @pl.when(c + 2 < _NRCHUNK)
        def _():
            rg_issue(c + 2, r0_v, sem0)

        @pl.when(c + 3 < _NRCHUNK)
        def _():
            wg_wait(c + 3)
            compute_m(c + 3)

        @pl.when(c + 4 < _NRCHUNK)
        def _():
            wg_issue(c + 4)

        rg_wait(c + 1, r1_v, sem1)
        store(c + 1, r1_v)
        return 0

    lax.fori_loop(0, _NRCHUNK // 2, row_body, 0)


def kernel(positions, pe_weight):
    b, s = positions.shape
    flat = positions.reshape(-1).astype(jnp.int32)
    part = _k1_partial_fp(flat)
    fp = _k2_combine_fp(part)
    g, lc = _k3_local_cumsum(flat, fp)
    out = _k4_map_and_gather(g, lc, pe_weight)
    return out.reshape(b, s, D_MODEL)
